# baseline (speedup 1.0000x reference)
import dataclasses
import functools

import jax
import jax.numpy as jnp
from jax import lax
from jax.experimental import pallas as pl
from jax.experimental.pallas import tpu as pltpu
from jax.experimental.pallas import tpu_sc as plsc

B, C, H, W = 2, 96, 384, 384
HW = H * W
N = B * HW
CB = 8
G = C // CB
NC, NS, L = 2, 16, 16
PPT = N // (NC * NS)
CH = 1536
NCHUNK = PPT // CH
ROWS_PER_TILE = PPT // W
IDX_ROWS = N // 128

_mesh = plsc.VectorSubcoreMesh(core_axis_name="c", subcore_axis_name="s")


def _compiler_params():
    cp = pltpu.CompilerParams(use_tc_tiling_on_sc=False)
    if "needs_layout_passes" in pltpu.CompilerParams.__dataclass_fields__:
        cp = dataclasses.replace(cp, needs_layout_passes=False)
    return cp


def _phase_a_body(fx_hbm, fy_hbm, idx_hbm, w_hbm, fxv, fyv, idxv, wv):
    c = lax.axis_index("c")
    s = lax.axis_index("s")
    base_pixel = c * HW + s * PPT
    base_irow = c * (HW // 128) + s * (PPT // 128)
    iota = lax.iota(jnp.int32, L)
    iotaf = iota.astype(jnp.float32)

    @pl.loop(0, NCHUNK)
    def chunk(i):
        poff = base_pixel + i * CH
        pltpu.sync_copy(fx_hbm.at[pl.ds(poff, CH)], fxv)
        pltpu.sync_copy(fy_hbm.at[pl.ds(poff, CH)], fyv)

        @pl.loop(0, CH // W)
        def rows(r):
            y0 = s * ROWS_PER_TILE + i * (CH // W) + r
            y0f = y0.astype(jnp.float32)

            @pl.loop(0, W // L)
            def cols(kk):
                q = r * W + kk * L
                xv = iotaf + (kk * L).astype(jnp.float32)
                X = xv + fxv[pl.ds(q, L)]
                Y = y0f + fyv[pl.ds(q, L)]
                xt = X.astype(jnp.int32).astype(jnp.float32)
                yt = Y.astype(jnp.int32).astype(jnp.float32)
                xf = jnp.where(X < xt, xt - 1.0, xt)
                yf = jnp.where(Y < yt, yt - 1.0, yt)
                valid = ((xf >= 0.0) & (xf + 1.0 <= W - 1)
                         & (yf >= 0.0) & (yf + 1.0 <= H - 1))
                vm = jnp.where(valid, 1.0, 0.0).astype(jnp.float32)
                wx1 = X - xf
                wx0 = 1.0 - wx1
                wy1 = Y - yf
                wy0 = 1.0 - wy1
                xi0 = jnp.clip(xf.astype(jnp.int32), 0, W - 1)
                xi1 = jnp.clip(xf.astype(jnp.int32) + 1, 0, W - 1)
                yi0 = jnp.clip(yf.astype(jnp.int32), 0, H - 1)
                yi1 = jnp.clip(yf.astype(jnp.int32) + 1, 0, H - 1)
                qrow = lax.shift_right_logical(q, 7)
                qent = lax.bitwise_and(q, 127)
                for k, (yy, xx, ww) in enumerate((
                        (yi0, xi0, wx0 * wy0 * vm),
                        (yi0, xi1, wx1 * wy0 * vm),
                        (yi1, xi0, wx0 * wy1 * vm),
                        (yi1, xi1, wx1 * wy1 * vm))):
                    idxv.at[k, qrow, pl.ds(qent, L)][...] = yy * W + xx
                    wv.at[k, pl.ds(q, L)][...] = ww

        pltpu.sync_copy(idxv, idx_hbm.at[:, pl.ds(base_irow + i * (CH // 128),
                                                  CH // 128), :])
        pltpu.sync_copy(wv, w_hbm.at[:, pl.ds(poff, CH)])


def _phase_b_body(src_hbm, init_hbm, idx_hbm, w_hbm, out_hbm,
                  slab, sv, iv, wv, ov):
    c = lax.axis_index("c")
    s = lax.axis_index("s")
    iota = lax.iota(jnp.int32, L)
    sel = (iota >= CB).astype(jnp.int32)
    chv = lax.bitwise_and(iota, CB - 1)
    kvecs = [jnp.full((L,), k, jnp.int32) for k in range(4)]

    @pl.loop(0, B * (G // NC))
    def slab_loop(t):
        b = lax.bitwise_and(t, 1)
        g = c * (G // NC) + lax.shift_right_logical(t, 1)
        pltpu.sync_copy(init_hbm.at[b, g, pl.ds(s * PPT, PPT), :],
                        slab.at[pl.ds(s * PPT, PPT), :])
        plsc.subcore_barrier()

        @pl.loop(0, NCHUNK)
        def chunk(i):
            poff = b * HW + s * PPT + i * CH
            rowoff = b * (HW // 128) + s * (PPT // 128) + i * (CH // 128)
            pltpu.sync_copy(src_hbm.at[b, g, pl.ds(s * PPT + i * CH, CH), :],
                            sv)
            pltpu.sync_copy(w_hbm.at[:, pl.ds(poff, CH)], wv)
            for k in range(4):
                pltpu.sync_copy(idx_hbm.at[k, pl.ds(rowoff, CH // 128), :],
                                iv.at[pl.ds(k * (CH // 128), CH // 128), :])

            for h in range(2):
                @pl.loop(0, CH // 4)
                def jloop(j):
                    q = sel + 2 * j
                    pp = q + h * (CH // 2)
                    srcv = plsc.load_gather(sv, [pp, chv])
                    prow = lax.shift_right_logical(q, 7)
                    pent = lax.bitwise_and(q, 127)
                    for k in range(4):
                        wk = plsc.load_gather(wv, [kvecs[k], pp])
                        plsc.store_scatter(
                            ov, [prow + k * (CH // 256), pent, chv],
                            srcv * wk)

                for k in range(4):
                    @pl.loop(0, CH // 256)
                    def scat(rr):
                        pltpu.sync_copy(
                            ov.at[k * (CH // 256) + rr],
                            slab.at[iv.at[k * (CH // 128)
                                          + h * (CH // 256) + rr]],
                            add=True)

        plsc.subcore_barrier()
        pltpu.sync_copy(slab.at[pl.ds(s * PPT, PPT), :],
                        out_hbm.at[b, g, pl.ds(s * PPT, PPT), :])
        plsc.subcore_barrier()


@jax.jit
def _forward_warp_sc(im0, flow, im1):
    fx = flow[..., 0].reshape(N)
    fy = flow[..., 1].reshape(N)
    src_r = im0.reshape(B, G, CB, HW).transpose(0, 1, 3, 2)
    init_r = im1.reshape(B, G, CB, HW).transpose(0, 1, 3, 2)

    phase_a = pl.kernel(
        _phase_a_body,
        out_type=(jax.ShapeDtypeStruct((4, IDX_ROWS, 128), jnp.int32),
                  jax.ShapeDtypeStruct((4, N), jnp.float32)),
        mesh=_mesh,
        scratch_types=[
            pltpu.VMEM((CH,), jnp.float32),
            pltpu.VMEM((CH,), jnp.float32),
            pltpu.VMEM((4, CH // 128, 128), jnp.int32),
            pltpu.VMEM((4, CH), jnp.float32),
        ],
        compiler_params=_compiler_params(),
    )
    idx_all, w_all = phase_a(fx, fy)

    phase_b = pl.kernel(
        _phase_b_body,
        out_type=jax.ShapeDtypeStruct((B, G, HW, CB), jnp.float32),
        mesh=_mesh,
        scratch_types=[
            pltpu.VMEM_SHARED((HW, CB), jnp.float32),
            pltpu.VMEM((CH, CB), jnp.float32),
            pltpu.VMEM((4 * (CH // 128), 128), jnp.int32),
            pltpu.VMEM((4, CH), jnp.float32),
            pltpu.VMEM((4 * (CH // 256), 128, CB), jnp.float32),
        ],
        compiler_params=_compiler_params(),
    )
    out_r = phase_b(src_r, init_r, idx_all, w_all)
    return out_r.transpose(0, 1, 3, 2).reshape(B, C, H, W)


def kernel(im0, flow, im1):
    return _forward_warp_sc(im0, flow, im1)
